# Initial kernel scaffold; baseline (speedup 1.0000x reference)
import jax
import jax.numpy as jnp
from jax.experimental import pallas as pl


def kernel(x, adj, W1, a1_src, a1_dst, W2, a2_src, a2_dst):
    raise NotImplementedError("write your pallas kernel here")


# speedup vs baseline: 2.0543x; 2.0543x over previous
import functools

import jax
import jax.numpy as jnp
from jax.experimental import pallas as pl
from jax.experimental.pallas import tpu as pltpu

_ALPHA = 0.2
_NEG = -9e15


def _proj_body(x_ref, w_ref, ab_ref, h_ref, esd_ref, esdt_ref):
    h = jnp.dot(x_ref[...], w_ref[...], preferred_element_type=jnp.float32)
    h_ref[...] = h
    esd = jnp.dot(h, ab_ref[...], preferred_element_type=jnp.float32)
    esd_ref[...] = esd
    esdt_ref[...] = esd.T


def _project(x, w, ab, tile):
    n, f = x.shape
    d = w.shape[1]
    m = ab.shape[1]
    return pl.pallas_call(
        _proj_body,
        grid=(n // tile,),
        in_specs=[
            pl.BlockSpec((tile, f), lambda i: (i, 0)),
            pl.BlockSpec((f, d), lambda i: (0, 0)),
            pl.BlockSpec((d, m), lambda i: (0, 0)),
        ],
        out_specs=[
            pl.BlockSpec((tile, d), lambda i: (i, 0)),
            pl.BlockSpec((tile, m), lambda i: (i, 0)),
            pl.BlockSpec((m, tile), lambda i: (0, i)),
        ],
        out_shape=[
            jax.ShapeDtypeStruct((n, d), jnp.float32),
            jax.ShapeDtypeStruct((n, m), jnp.float32),
            jax.ShapeDtypeStruct((m, n), jnp.float32),
        ],
    )(x, w, ab)


def _att_body(adj_ref, esd_ref, esdt_ref, h_ref, o_ref, *, heads, hdim, relu):
    adj = adj_ref[...]
    for k in range(heads):
        s = esd_ref[:, k:k + 1]
        d = esdt_ref[heads + k:heads + k + 1, :]
        e = s + d
        e = jnp.where(e >= 0, e, _ALPHA * e)
        e = jnp.where(adj > 0, e, _NEG)
        m = jnp.max(e, axis=1, keepdims=True)
        p = jnp.exp(e - m)
        z = jnp.sum(p, axis=1, keepdims=True)
        acc = jnp.dot(p, h_ref[:, k * hdim:(k + 1) * hdim],
                      preferred_element_type=jnp.float32) / z
        if relu:
            acc = jnp.maximum(acc, 0.0)
        o_ref[:, k * hdim:(k + 1) * hdim] = acc


def _attention(adj, esd, esdt, h, heads, hdim, relu, tile):
    n = adj.shape[0]
    d = h.shape[1]
    m = esd.shape[1]
    body = functools.partial(_att_body, heads=heads, hdim=hdim, relu=relu)
    return pl.pallas_call(
        body,
        grid=(n // tile,),
        in_specs=[
            pl.BlockSpec((tile, n), lambda i: (i, 0)),
            pl.BlockSpec((tile, m), lambda i: (i, 0)),
            pl.BlockSpec((m, n), lambda i: (0, 0)),
            pl.BlockSpec((n, d), lambda i: (0, 0)),
        ],
        out_specs=pl.BlockSpec((tile, d), lambda i: (i, 0)),
        out_shape=jax.ShapeDtypeStruct((n, d), jnp.float32),
        compiler_params=pltpu.CompilerParams(
            dimension_semantics=("arbitrary",)),
    )(adj, esd, esdt, h)


def kernel(x, adj, W1, a1_src, a1_dst, W2, a2_src, a2_dst):
    K, F_IN, H = W1.shape
    C = W2.shape[2]
    N = x.shape[0]

    w1c = jnp.transpose(W1, (1, 0, 2)).reshape(F_IN, K * H)
    eye_k = jnp.eye(K, dtype=jnp.float32)
    A1 = jnp.einsum('ko,kj->koj', a1_src, eye_k).reshape(K * H, K)
    B1 = jnp.einsum('ko,kj->koj', a1_dst, eye_k).reshape(K * H, K)
    ab1 = jnp.concatenate([A1, B1], axis=1)
    w2c = W2.reshape(K * H, C)
    ab2 = jnp.concatenate([a2_src.T, a2_dst.T], axis=1)

    h1, esd1, esd1t = _project(x, w1c, ab1, tile=512)
    o1 = _attention(adj, esd1, esd1t, h1, heads=K, hdim=H, relu=True,
                    tile=256)
    h2, esd2, esd2t = _project(o1, w2c, ab2, tile=512)
    out = _attention(adj, esd2, esd2t, h2, heads=1, hdim=C, relu=False,
                     tile=256)
    return out


# speedup vs baseline: 2.6778x; 1.3035x over previous
import functools

import jax
import jax.numpy as jnp
from jax.experimental import pallas as pl
from jax.experimental.pallas import tpu as pltpu

_ALPHA = 0.2


def _proj_body(x_ref, w_ref, ab_ref, h_ref, rowfac_ref, colfac_ref):
    h = jnp.dot(x_ref[...], w_ref[...], preferred_element_type=jnp.float32)
    h_ref[...] = h
    g = jnp.dot(h, ab_ref[...], preferred_element_type=jnp.float32)
    m = g.shape[1] // 2
    gs, gd = g[:, :m], g[:, m:]
    rowfac_ref[...] = jnp.exp(jnp.concatenate([gs, _ALPHA * gs], axis=1))
    colfac_ref[...] = jnp.exp(jnp.concatenate([gd, _ALPHA * gd], axis=1)).T


def _project(x, w, ab, tile):
    n, f = x.shape
    d = w.shape[1]
    m2 = ab.shape[1]
    return pl.pallas_call(
        _proj_body,
        grid=(n // tile,),
        in_specs=[
            pl.BlockSpec((tile, f), lambda i: (i, 0)),
            pl.BlockSpec((f, d), lambda i: (0, 0)),
            pl.BlockSpec((d, m2), lambda i: (0, 0)),
        ],
        out_specs=[
            pl.BlockSpec((tile, d), lambda i: (i, 0)),
            pl.BlockSpec((tile, m2), lambda i: (i, 0)),
            pl.BlockSpec((m2, tile), lambda i: (0, i)),
        ],
        out_shape=[
            jax.ShapeDtypeStruct((n, d), jnp.float32),
            jax.ShapeDtypeStruct((n, m2), jnp.float32),
            jax.ShapeDtypeStruct((m2, n), jnp.float32),
        ],
    )(x, w, ab)


def _att_body(adj_ref, rowfac_ref, colfac_ref, h_ref, o_ref,
              *, heads, hdim, relu):
    adj = adj_ref[...]
    hmean = jnp.mean(h_ref[...], axis=0, keepdims=True)
    for k in range(heads):
        es = rowfac_ref[:, k:k + 1]
        fs = rowfac_ref[:, heads + k:heads + k + 1]
        ed = colfac_ref[k:k + 1, :]
        fd = colfac_ref[heads + k:heads + k + 1, :]
        w = jnp.maximum(es * ed, fs * fd) * adj
        z = jnp.sum(w, axis=1, keepdims=True)
        acc = jnp.dot(w, h_ref[:, k * hdim:(k + 1) * hdim],
                      preferred_element_type=jnp.float32)
        acc = jnp.where(z > 0, acc / z, hmean[:, k * hdim:(k + 1) * hdim])
        if relu:
            acc = jnp.maximum(acc, 0.0)
        o_ref[:, k * hdim:(k + 1) * hdim] = acc


def _attention(adj, rowfac, colfac, h, heads, hdim, relu, tile):
    n = adj.shape[0]
    d = h.shape[1]
    m2 = rowfac.shape[1]
    body = functools.partial(_att_body, heads=heads, hdim=hdim, relu=relu)
    return pl.pallas_call(
        body,
        grid=(n // tile,),
        in_specs=[
            pl.BlockSpec((tile, n), lambda i: (i, 0)),
            pl.BlockSpec((tile, m2), lambda i: (i, 0)),
            pl.BlockSpec((m2, n), lambda i: (0, 0)),
            pl.BlockSpec((n, d), lambda i: (0, 0)),
        ],
        out_specs=pl.BlockSpec((tile, d), lambda i: (i, 0)),
        out_shape=jax.ShapeDtypeStruct((n, d), jnp.float32),
        compiler_params=pltpu.CompilerParams(
            dimension_semantics=("arbitrary",)),
    )(adj, rowfac, colfac, h)


def kernel(x, adj, W1, a1_src, a1_dst, W2, a2_src, a2_dst):
    K, F_IN, H = W1.shape
    C = W2.shape[2]

    w1c = jnp.transpose(W1, (1, 0, 2)).reshape(F_IN, K * H)
    eye_k = jnp.eye(K, dtype=jnp.float32)
    A1 = jnp.einsum('ko,kj->koj', a1_src, eye_k).reshape(K * H, K)
    B1 = jnp.einsum('ko,kj->koj', a1_dst, eye_k).reshape(K * H, K)
    ab1 = jnp.concatenate([A1, B1], axis=1)
    w2c = W2.reshape(K * H, C)
    ab2 = jnp.concatenate([a2_src.T, a2_dst.T], axis=1)

    h1, rf1, cf1 = _project(x, w1c, ab1, tile=512)
    o1 = _attention(adj, rf1, cf1, h1, heads=K, hdim=H, relu=True, tile=256)
    h2, rf2, cf2 = _project(o1, w2c, ab2, tile=512)
    out = _attention(adj, rf2, cf2, h2, heads=1, hdim=C, relu=False, tile=256)
    return out


# speedup vs baseline: 2.7739x; 1.0359x over previous
import functools

import jax
import jax.numpy as jnp
from jax.experimental import pallas as pl
from jax.experimental.pallas import tpu as pltpu

_ALPHA = 0.2


def _proj_body(x_ref, w_ref, ab_ref, h_ref, rowfac_ref, colfac_ref):
    h = jnp.dot(x_ref[...], w_ref[...], preferred_element_type=jnp.float32)
    h_ref[...] = h
    g = jnp.dot(h, ab_ref[...], preferred_element_type=jnp.float32)
    m = g.shape[1] // 2
    gs, gd = g[:, :m], g[:, m:]
    rowfac_ref[...] = jnp.exp((_ALPHA - 1.0) * gs)
    colfac_ref[...] = jnp.exp(jnp.concatenate([gd, _ALPHA * gd], axis=1)).T


def _project(x, w, ab, tile):
    n, f = x.shape
    d = w.shape[1]
    m2 = ab.shape[1]
    return pl.pallas_call(
        _proj_body,
        grid=(n // tile,),
        in_specs=[
            pl.BlockSpec((tile, f), lambda i: (i, 0)),
            pl.BlockSpec((f, d), lambda i: (0, 0)),
            pl.BlockSpec((d, m2), lambda i: (0, 0)),
        ],
        out_specs=[
            pl.BlockSpec((tile, d), lambda i: (i, 0)),
            pl.BlockSpec((tile, m2 // 2), lambda i: (i, 0)),
            pl.BlockSpec((m2, tile), lambda i: (0, i)),
        ],
        out_shape=[
            jax.ShapeDtypeStruct((n, d), jnp.float32),
            jax.ShapeDtypeStruct((n, m2 // 2), jnp.float32),
            jax.ShapeDtypeStruct((m2, n), jnp.float32),
        ],
    )(x, w, ab)


def _att_body(adj_ref, rowfac_ref, colfac_ref, h_ref, o_ref,
              *, heads, hdim, relu):
    adj = adj_ref[...]
    hmean = jnp.mean(h_ref[...], axis=0, keepdims=True)
    for k in range(heads):
        r = rowfac_ref[:, k:k + 1]
        ed = colfac_ref[k:k + 1, :]
        fd = colfac_ref[heads + k:heads + k + 1, :]
        w = jnp.maximum(ed, r * fd) * adj
        z = jnp.sum(w, axis=1, keepdims=True)
        acc = jnp.dot(w, h_ref[:, k * hdim:(k + 1) * hdim],
                      preferred_element_type=jnp.float32)
        acc = jnp.where(z > 0, acc / z, hmean[:, k * hdim:(k + 1) * hdim])
        if relu:
            acc = jnp.maximum(acc, 0.0)
        o_ref[:, k * hdim:(k + 1) * hdim] = acc


def _attention(adj, rowfac, colfac, h, heads, hdim, relu, tile):
    n = adj.shape[0]
    d = h.shape[1]
    m2 = colfac.shape[0]
    mh = rowfac.shape[1]
    body = functools.partial(_att_body, heads=heads, hdim=hdim, relu=relu)
    return pl.pallas_call(
        body,
        grid=(n // tile,),
        in_specs=[
            pl.BlockSpec((tile, n), lambda i: (i, 0)),
            pl.BlockSpec((tile, mh), lambda i: (i, 0)),
            pl.BlockSpec((m2, n), lambda i: (0, 0)),
            pl.BlockSpec((n, d), lambda i: (0, 0)),
        ],
        out_specs=pl.BlockSpec((tile, d), lambda i: (i, 0)),
        out_shape=jax.ShapeDtypeStruct((n, d), jnp.float32),
        compiler_params=pltpu.CompilerParams(
            dimension_semantics=("parallel",)),
    )(adj, rowfac, colfac, h)


def kernel(x, adj, W1, a1_src, a1_dst, W2, a2_src, a2_dst):
    K, F_IN, H = W1.shape
    C = W2.shape[2]

    w1c = jnp.transpose(W1, (1, 0, 2)).reshape(F_IN, K * H)
    eye_k = jnp.eye(K, dtype=jnp.float32)
    A1 = jnp.einsum('ko,kj->koj', a1_src, eye_k).reshape(K * H, K)
    B1 = jnp.einsum('ko,kj->koj', a1_dst, eye_k).reshape(K * H, K)
    ab1 = jnp.concatenate([A1, B1], axis=1)
    w2c = W2.reshape(K * H, C)
    ab2 = jnp.concatenate([a2_src.T, a2_dst.T], axis=1)

    h1, rf1, cf1 = _project(x, w1c, ab1, tile=512)
    o1 = _attention(adj, rf1, cf1, h1, heads=K, hdim=H, relu=True, tile=256)
    h2, rf2, cf2 = _project(o1, w2c, ab2, tile=512)
    out = _attention(adj, rf2, cf2, h2, heads=1, hdim=C, relu=False, tile=256)
    return out


# speedup vs baseline: 3.8382x; 1.3837x over previous
import functools

import jax
import jax.numpy as jnp
from jax.experimental import pallas as pl
from jax.experimental.pallas import tpu as pltpu

_ALPHA = 0.2
_PAD = 8


def _proj_body(x_ref, w_ref, ab_ref, h_ref, hpack_ref, rowfac_ref,
               colfac_ref, hsum_ref, *, heads, hdim):
    i = pl.program_id(0)
    h = jnp.dot(x_ref[...], w_ref[...], preferred_element_type=jnp.float32)
    h_ref[...] = h
    tile = h.shape[0]
    pieces = []
    ones = jnp.ones((tile, 1), dtype=jnp.float32)
    zeros = jnp.zeros((tile, _PAD - 1), dtype=jnp.float32)
    for k in range(heads):
        pieces += [h[:, k * hdim:(k + 1) * hdim], ones, zeros]
    hpack_ref[...] = jnp.concatenate(pieces, axis=1).astype(jnp.bfloat16)
    g = jnp.dot(h, ab_ref[...], preferred_element_type=jnp.float32)
    m = g.shape[1] // 2
    gs, gd = g[:, :m], g[:, m:]
    rowfac_ref[...] = jnp.exp((_ALPHA - 1.0) * gs).astype(jnp.bfloat16)
    colfac_ref[...] = jnp.exp(
        jnp.concatenate([gd, _ALPHA * gd], axis=1)).astype(jnp.bfloat16).T

    @pl.when(i == 0)
    def _init():
        hsum_ref[...] = jnp.zeros_like(hsum_ref)

    hsum_ref[...] += jnp.sum(h, axis=0, keepdims=True)


def _project(x, w, ab, heads, hdim, tile):
    n, f = x.shape
    d = w.shape[1]
    m2 = ab.shape[1]
    dp = heads * (hdim + _PAD)
    body = functools.partial(_proj_body, heads=heads, hdim=hdim)
    return pl.pallas_call(
        body,
        grid=(n // tile,),
        in_specs=[
            pl.BlockSpec((tile, f), lambda i: (i, 0)),
            pl.BlockSpec((f, d), lambda i: (0, 0)),
            pl.BlockSpec((d, m2), lambda i: (0, 0)),
        ],
        out_specs=[
            pl.BlockSpec((tile, d), lambda i: (i, 0)),
            pl.BlockSpec((tile, dp), lambda i: (i, 0)),
            pl.BlockSpec((tile, m2 // 2), lambda i: (i, 0)),
            pl.BlockSpec((m2, tile), lambda i: (0, i)),
            pl.BlockSpec((1, d), lambda i: (0, 0)),
        ],
        out_shape=[
            jax.ShapeDtypeStruct((n, d), jnp.float32),
            jax.ShapeDtypeStruct((n, dp), jnp.bfloat16),
            jax.ShapeDtypeStruct((n, m2 // 2), jnp.bfloat16),
            jax.ShapeDtypeStruct((m2, n), jnp.bfloat16),
            jax.ShapeDtypeStruct((1, d), jnp.float32),
        ],
    )(x, w, ab)


def _att_body(adj_ref, rowfac_ref, colfac_ref, hpack_ref, hsum_ref, o_ref,
              *, heads, hdim, relu):
    adjb = adj_ref[...].astype(jnp.bfloat16)
    n_src = adj_ref.shape[1]
    stride = hdim + _PAD
    for k in range(heads):
        r = rowfac_ref[:, k:k + 1]
        ed = colfac_ref[k:k + 1, :]
        fd = colfac_ref[heads + k:heads + k + 1, :]
        w = jnp.maximum(ed, r * fd) * adjb
        acc = jnp.dot(w, hpack_ref[:, k * stride:k * stride + hdim + 1],
                      preferred_element_type=jnp.float32)
        num, z = acc[:, :hdim], acc[:, hdim:hdim + 1]
        hmean = hsum_ref[:, k * hdim:(k + 1) * hdim] * (1.0 / n_src)
        out = jnp.where(z > 0, num / z, hmean)
        if relu:
            out = jnp.maximum(out, 0.0)
        o_ref[:, k * hdim:(k + 1) * hdim] = out


def _attention(adj, rowfac, colfac, hpack, hsum, heads, hdim, relu, tile):
    n = adj.shape[0]
    m2 = colfac.shape[0]
    mh = rowfac.shape[1]
    dp = hpack.shape[1]
    d = heads * hdim
    body = functools.partial(_att_body, heads=heads, hdim=hdim, relu=relu)
    return pl.pallas_call(
        body,
        grid=(n // tile,),
        in_specs=[
            pl.BlockSpec((tile, n), lambda i: (i, 0)),
            pl.BlockSpec((tile, mh), lambda i: (i, 0)),
            pl.BlockSpec((m2, n), lambda i: (0, 0)),
            pl.BlockSpec((n, dp), lambda i: (0, 0)),
            pl.BlockSpec((1, d), lambda i: (0, 0)),
        ],
        out_specs=pl.BlockSpec((tile, d), lambda i: (i, 0)),
        out_shape=jax.ShapeDtypeStruct((n, d), jnp.float32),
        compiler_params=pltpu.CompilerParams(
            dimension_semantics=("parallel",)),
    )(adj, rowfac, colfac, hpack, hsum)


def kernel(x, adj, W1, a1_src, a1_dst, W2, a2_src, a2_dst):
    K, F_IN, H = W1.shape
    C = W2.shape[2]

    w1c = jnp.transpose(W1, (1, 0, 2)).reshape(F_IN, K * H)
    eye_k = jnp.eye(K, dtype=jnp.float32)
    A1 = jnp.einsum('ko,kj->koj', a1_src, eye_k).reshape(K * H, K)
    B1 = jnp.einsum('ko,kj->koj', a1_dst, eye_k).reshape(K * H, K)
    ab1 = jnp.concatenate([A1, B1], axis=1)
    w2c = W2.reshape(K * H, C)
    ab2 = jnp.concatenate([a2_src.T, a2_dst.T], axis=1)

    _, hp1, rf1, cf1, hs1 = _project(x, w1c, ab1, heads=K, hdim=H, tile=512)
    o1 = _attention(adj, rf1, cf1, hp1, hs1, heads=K, hdim=H, relu=True,
                    tile=256)
    _, hp2, rf2, cf2, hs2 = _project(o1, w2c, ab2, heads=1, hdim=C, tile=512)
    out = _attention(adj, rf2, cf2, hp2, hs2, heads=1, hdim=C, relu=False,
                     tile=256)
    return out


# speedup vs baseline: 4.1358x; 1.0775x over previous
import functools

import jax
import jax.numpy as jnp
from jax.experimental import pallas as pl
from jax.experimental.pallas import tpu as pltpu

_ALPHA = 0.2
_PAD = 8


def _proj_body(x_ref, w_ref, ab_ref, h_ref, hpack_ref, rowfac_ref,
               colfac_ref, hsum_ref, *, heads, hdim):
    i = pl.program_id(0)
    h = jnp.dot(x_ref[...], w_ref[...], preferred_element_type=jnp.float32)
    h_ref[...] = h
    tile = h.shape[0]
    pieces = []
    ones = jnp.ones((tile, 1), dtype=jnp.float32)
    zeros = jnp.zeros((tile, _PAD - 1), dtype=jnp.float32)
    for k in range(heads):
        pieces += [h[:, k * hdim:(k + 1) * hdim], ones, zeros]
    hpack_ref[...] = jnp.concatenate(pieces, axis=1).astype(jnp.bfloat16)
    g = jnp.dot(h, ab_ref[...], preferred_element_type=jnp.float32)
    m = g.shape[1] // 2
    gs, gd = g[:, :m], g[:, m:]
    rowfac_ref[...] = jnp.exp((_ALPHA - 1.0) * gs).astype(jnp.bfloat16)
    colfac_ref[...] = jnp.exp(
        jnp.concatenate([gd, _ALPHA * gd], axis=1)).astype(jnp.bfloat16).T

    @pl.when(i == 0)
    def _init():
        hsum_ref[...] = jnp.zeros_like(hsum_ref)

    hsum_ref[...] += jnp.sum(h, axis=0, keepdims=True)


def _project(x, w, ab, heads, hdim, tile):
    n, f = x.shape
    d = w.shape[1]
    m2 = ab.shape[1]
    dp = heads * (hdim + _PAD)
    body = functools.partial(_proj_body, heads=heads, hdim=hdim)
    return pl.pallas_call(
        body,
        grid=(n // tile,),
        in_specs=[
            pl.BlockSpec((tile, f), lambda i: (i, 0)),
            pl.BlockSpec((f, d), lambda i: (0, 0)),
            pl.BlockSpec((d, m2), lambda i: (0, 0)),
        ],
        out_specs=[
            pl.BlockSpec((tile, d), lambda i: (i, 0)),
            pl.BlockSpec((tile, dp), lambda i: (i, 0)),
            pl.BlockSpec((tile, m2 // 2), lambda i: (i, 0)),
            pl.BlockSpec((m2, tile), lambda i: (0, i)),
            pl.BlockSpec((1, d), lambda i: (0, 0)),
        ],
        out_shape=[
            jax.ShapeDtypeStruct((n, d), jnp.float32),
            jax.ShapeDtypeStruct((n, dp), jnp.bfloat16),
            jax.ShapeDtypeStruct((n, m2 // 2), jnp.bfloat16),
            jax.ShapeDtypeStruct((m2, n), jnp.bfloat16),
            jax.ShapeDtypeStruct((1, d), jnp.float32),
        ],
    )(x, w, ab)


def _att_body(adj_ref, rowfac_ref, colfac_ref, hpack_ref, hsum_ref, o_ref,
              *maybe_mask_ref, heads, hdim, relu):
    adjb = adj_ref[...].astype(jnp.bfloat16)
    if maybe_mask_ref:
        maybe_mask_ref[0][...] = adjb.astype(jnp.float8_e5m2)
    n_src = adj_ref.shape[1]
    stride = hdim + _PAD
    for k in range(heads):
        r = rowfac_ref[:, k:k + 1]
        ed = colfac_ref[k:k + 1, :]
        fd = colfac_ref[heads + k:heads + k + 1, :]
        w = jnp.maximum(ed, r * fd) * adjb
        acc = jnp.dot(w, hpack_ref[:, k * stride:k * stride + hdim + 1],
                      preferred_element_type=jnp.float32)
        num, z = acc[:, :hdim], acc[:, hdim:hdim + 1]
        hmean = hsum_ref[:, k * hdim:(k + 1) * hdim] * (1.0 / n_src)
        out = jnp.where(z > 0, num / z, hmean)
        if relu:
            out = jnp.maximum(out, 0.0)
        o_ref[:, k * hdim:(k + 1) * hdim] = out


def _attention(adj, rowfac, colfac, hpack, hsum, heads, hdim, relu, tile,
               emit_mask=False):
    n = adj.shape[0]
    m2 = colfac.shape[0]
    mh = rowfac.shape[1]
    dp = hpack.shape[1]
    d = heads * hdim
    body = functools.partial(_att_body, heads=heads, hdim=hdim, relu=relu)
    out_specs = [pl.BlockSpec((tile, d), lambda i: (i, 0))]
    out_shape = [jax.ShapeDtypeStruct((n, d), jnp.float32)]
    if emit_mask:
        out_specs.append(pl.BlockSpec((tile, n), lambda i: (i, 0)))
        out_shape.append(jax.ShapeDtypeStruct((n, n), jnp.float8_e5m2))
    res = pl.pallas_call(
        body,
        grid=(n // tile,),
        in_specs=[
            pl.BlockSpec((tile, n), lambda i: (i, 0)),
            pl.BlockSpec((tile, mh), lambda i: (i, 0)),
            pl.BlockSpec((m2, n), lambda i: (0, 0)),
            pl.BlockSpec((n, dp), lambda i: (0, 0)),
            pl.BlockSpec((1, d), lambda i: (0, 0)),
        ],
        out_specs=out_specs,
        out_shape=out_shape,
        compiler_params=pltpu.CompilerParams(
            dimension_semantics=("parallel",)),
    )(adj, rowfac, colfac, hpack, hsum)
    return res if emit_mask else (res[0], None)


def kernel(x, adj, W1, a1_src, a1_dst, W2, a2_src, a2_dst):
    K, F_IN, H = W1.shape
    C = W2.shape[2]

    w1c = jnp.transpose(W1, (1, 0, 2)).reshape(F_IN, K * H)
    eye_k = jnp.eye(K, dtype=jnp.float32)
    A1 = jnp.einsum('ko,kj->koj', a1_src, eye_k).reshape(K * H, K)
    B1 = jnp.einsum('ko,kj->koj', a1_dst, eye_k).reshape(K * H, K)
    ab1 = jnp.concatenate([A1, B1], axis=1)
    w2c = W2.reshape(K * H, C)
    ab2 = jnp.concatenate([a2_src.T, a2_dst.T], axis=1)

    _, hp1, rf1, cf1, hs1 = _project(x, w1c, ab1, heads=K, hdim=H, tile=512)
    o1, mask8 = _attention(adj, rf1, cf1, hp1, hs1, heads=K, hdim=H,
                           relu=True, tile=256, emit_mask=True)
    _, hp2, rf2, cf2, hs2 = _project(o1, w2c, ab2, heads=1, hdim=C, tile=512)
    out, _ = _attention(mask8, rf2, cf2, hp2, hs2, heads=1, hdim=C,
                        relu=False, tile=256)
    return out


# speedup vs baseline: 4.5283x; 1.0949x over previous
import functools

import jax
import jax.numpy as jnp
from jax.experimental import pallas as pl
from jax.experimental.pallas import tpu as pltpu

_ALPHA = 0.2
_PAD = 8


def _proj_body(x_ref, w_ref, ab_ref, h_ref, hpack_ref, rowfac_ref,
               colfac_ref, hsum_ref, *, heads, hdim):
    i = pl.program_id(0)
    h = jnp.dot(x_ref[...], w_ref[...], preferred_element_type=jnp.float32)
    h_ref[...] = h
    tile = h.shape[0]
    pieces = []
    ones = jnp.ones((tile, 1), dtype=jnp.float32)
    zeros = jnp.zeros((tile, _PAD - 1), dtype=jnp.float32)
    for k in range(heads):
        pieces += [h[:, k * hdim:(k + 1) * hdim], ones, zeros]
    hpack_ref[...] = jnp.concatenate(pieces, axis=1).astype(jnp.bfloat16)
    g = jnp.dot(h, ab_ref[...], preferred_element_type=jnp.float32)
    m = g.shape[1] // 2
    gs, gd = g[:, :m], g[:, m:]
    rowfac_ref[...] = jnp.exp((_ALPHA - 1.0) * gs).astype(jnp.bfloat16)
    colfac_ref[...] = jnp.exp(
        jnp.concatenate([gd, _ALPHA * gd], axis=1)).astype(jnp.bfloat16).T

    @pl.when(i == 0)
    def _init():
        hsum_ref[...] = jnp.zeros_like(hsum_ref)

    hsum_ref[...] += jnp.sum(h, axis=0, keepdims=True)


def _project(x, w, ab, heads, hdim, tile):
    n, f = x.shape
    d = w.shape[1]
    m2 = ab.shape[1]
    dp = heads * (hdim + _PAD)
    body = functools.partial(_proj_body, heads=heads, hdim=hdim)
    return pl.pallas_call(
        body,
        grid=(n // tile,),
        in_specs=[
            pl.BlockSpec((tile, f), lambda i: (i, 0)),
            pl.BlockSpec((f, d), lambda i: (0, 0)),
            pl.BlockSpec((d, m2), lambda i: (0, 0)),
        ],
        out_specs=[
            pl.BlockSpec((tile, d), lambda i: (i, 0)),
            pl.BlockSpec((tile, dp), lambda i: (i, 0)),
            pl.BlockSpec((tile, m2 // 2), lambda i: (i, 0)),
            pl.BlockSpec((m2, tile), lambda i: (0, i)),
            pl.BlockSpec((1, d), lambda i: (0, 0)),
        ],
        out_shape=[
            jax.ShapeDtypeStruct((n, d), jnp.float32),
            jax.ShapeDtypeStruct((n, dp), jnp.bfloat16),
            jax.ShapeDtypeStruct((n, m2 // 2), jnp.bfloat16),
            jax.ShapeDtypeStruct((m2, n), jnp.bfloat16),
            jax.ShapeDtypeStruct((1, d), jnp.float32),
        ],
    )(x, w, ab)


def _att_body(adj_ref, rowfac_ref, colfac_ref, hpack_ref, hsum_ref, o_ref,
              *maybe_mask_ref, heads, hdim, relu):
    adjb = adj_ref[...].astype(jnp.bfloat16)
    if maybe_mask_ref:
        maybe_mask_ref[0][...] = adjb.astype(jnp.float8_e5m2)
    n_src = adj_ref.shape[1]
    stride = hdim + _PAD
    for k in range(heads):
        r = rowfac_ref[:, k:k + 1]
        ed = colfac_ref[k:k + 1, :]
        fd = colfac_ref[heads + k:heads + k + 1, :]
        w = jnp.maximum(ed, r * fd) * adjb
        acc = jnp.dot(w, hpack_ref[:, k * stride:k * stride + hdim + 1],
                      preferred_element_type=jnp.float32)
        num, z = acc[:, :hdim], acc[:, hdim:hdim + 1]
        hmean = hsum_ref[:, k * hdim:(k + 1) * hdim] * (1.0 / n_src)
        out = jnp.where(z > 0, num / z, hmean)
        if relu:
            out = jnp.maximum(out, 0.0)
        o_ref[:, k * hdim:(k + 1) * hdim] = out


def _attention(adj, rowfac, colfac, hpack, hsum, heads, hdim, relu, tile,
               emit_mask=False):
    n = adj.shape[0]
    m2 = colfac.shape[0]
    mh = rowfac.shape[1]
    dp = hpack.shape[1]
    d = heads * hdim
    body = functools.partial(_att_body, heads=heads, hdim=hdim, relu=relu)
    out_specs = [pl.BlockSpec((tile, d), lambda i: (i, 0))]
    out_shape = [jax.ShapeDtypeStruct((n, d), jnp.float32)]
    if emit_mask:
        out_specs.append(pl.BlockSpec((tile, n), lambda i: (i, 0)))
        out_shape.append(jax.ShapeDtypeStruct((n, n), jnp.float8_e5m2))
    res = pl.pallas_call(
        body,
        grid=(n // tile,),
        in_specs=[
            pl.BlockSpec((tile, n), lambda i: (i, 0)),
            pl.BlockSpec((tile, mh), lambda i: (i, 0)),
            pl.BlockSpec((m2, n), lambda i: (0, 0)),
            pl.BlockSpec((n, dp), lambda i: (0, 0)),
            pl.BlockSpec((1, d), lambda i: (0, 0)),
        ],
        out_specs=out_specs,
        out_shape=out_shape,
        compiler_params=pltpu.CompilerParams(
            dimension_semantics=("parallel",)),
    )(adj, rowfac, colfac, hpack, hsum)
    return res if emit_mask else (res[0], None)


def kernel(x, adj, W1, a1_src, a1_dst, W2, a2_src, a2_dst):
    K, F_IN, H = W1.shape
    C = W2.shape[2]

    w1c = jnp.transpose(W1, (1, 0, 2)).reshape(F_IN, K * H)
    eye_k = jnp.eye(K, dtype=jnp.float32)
    A1 = jnp.einsum('ko,kj->koj', a1_src, eye_k).reshape(K * H, K)
    B1 = jnp.einsum('ko,kj->koj', a1_dst, eye_k).reshape(K * H, K)
    ab1 = jnp.concatenate([A1, B1], axis=1)
    w2c = W2.reshape(K * H, C)
    ab2 = jnp.concatenate([a2_src.T, a2_dst.T], axis=1)

    _, hp1, rf1, cf1, hs1 = _project(x, w1c, ab1, heads=K, hdim=H, tile=512)
    o1, mask8 = _attention(adj, rf1, cf1, hp1, hs1, heads=K, hdim=H,
                           relu=True, tile=512, emit_mask=True)
    _, hp2, rf2, cf2, hs2 = _project(o1, w2c, ab2, heads=1, hdim=C, tile=512)
    out, _ = _attention(mask8, rf2, cf2, hp2, hs2, heads=1, hdim=C,
                        relu=False, tile=512)
    return out
